# initial kernel scaffold (unmeasured)
import jax
import jax.numpy as jnp
from jax import lax
from jax.experimental import pallas as pl
from jax.experimental.pallas import tpu as pltpu


def kernel(
    x,
):
    def body(*refs):
        pass

    out_shape = jax.ShapeDtypeStruct(..., jnp.float32)
    return pl.pallas_call(body, out_shape=out_shape)(...)


# baseline (device time: 422797 ns/iter reference)
import jax
import jax.numpy as jnp
from jax import lax
from jax.experimental import pallas as pl
from jax.experimental.pallas import tpu as pltpu

K = 16


def kernel(x):
    m, n = x.shape
    assert m % K == 0
    c = m // K
    xb = x.astype(jnp.bfloat16)

    def body(xb_ref, out_ref, recv_ref, send_sems, recv_sems,
             xv, rv, ov, cp_sems):
        mx = lax.axis_index("x")
        my = lax.axis_index("y")
        mz = lax.axis_index("z")
        peer = (1 - mx, my, mz)

        barrier = pltpu.get_barrier_semaphore()
        pl.semaphore_signal(barrier, inc=1, device_id=peer,
                            device_id_type=pl.DeviceIdType.MESH)
        pl.semaphore_wait(barrier, 1)

        rdmas = []
        for k in range(K):
            rdma = pltpu.make_async_remote_copy(
                src_ref=xb_ref.at[pl.ds(k * c, c), :],
                dst_ref=recv_ref.at[pl.ds(k * c, c), :],
                send_sem=send_sems.at[k],
                recv_sem=recv_sems.at[k],
                device_id=peer,
                device_id_type=pl.DeviceIdType.MESH,
            )
            rdma.start()
            rdmas.append(rdma)

        for k in range(K):
            rdmas[k].wait_recv()
            cx = pltpu.make_async_copy(
                xb_ref.at[pl.ds(k * c, c), :], xv, cp_sems.at[0])
            cr = pltpu.make_async_copy(
                recv_ref.at[pl.ds(k * c, c), :], rv, cp_sems.at[1])
            cx.start()
            cr.start()
            cx.wait()
            cr.wait()
            ov[...] = xv[...] + rv[...]
            co = pltpu.make_async_copy(
                ov, out_ref.at[pl.ds(k * c, c), :], cp_sems.at[2])
            co.start()
            co.wait()

        for k in range(K):
            rdmas[k].wait_send()

    out, _recv = pl.pallas_call(
        body,
        out_shape=(
            jax.ShapeDtypeStruct((m, n), jnp.bfloat16),
            jax.ShapeDtypeStruct((m, n), jnp.bfloat16),
        ),
        in_specs=[pl.BlockSpec(memory_space=pl.ANY)],
        out_specs=(
            pl.BlockSpec(memory_space=pl.ANY),
            pl.BlockSpec(memory_space=pl.ANY),
        ),
        scratch_shapes=[
            pltpu.SemaphoreType.DMA((K,)),
            pltpu.SemaphoreType.DMA((K,)),
            pltpu.MemorySpace.VMEM((c, n), jnp.bfloat16),
            pltpu.MemorySpace.VMEM((c, n), jnp.bfloat16),
            pltpu.MemorySpace.VMEM((c, n), jnp.bfloat16),
            pltpu.SemaphoreType.DMA((3,)),
        ],
        compiler_params=pltpu.CompilerParams(collective_id=0),
    )(xb)
    return out


# device time: 257170 ns/iter; 1.6440x vs baseline; 1.6440x over previous
import jax
import jax.numpy as jnp
from jax import lax
from jax.experimental import pallas as pl
from jax.experimental.pallas import tpu as pltpu

K = 32
J = K // 2


def kernel(x):
    m, n = x.shape
    assert m % K == 0
    c = m // K
    xb = x.astype(jnp.bfloat16)

    def body(xb_ref, out_ref, recv_ref,
             xs_sems, xr_sems, fs_sems, yr_sems,
             xv, rv, ov, cp_sems):
        mx = lax.axis_index("x")
        my = lax.axis_index("y")
        mz = lax.axis_index("z")
        xpeer = (1 - mx, my, mz)
        ypeer = (mx, 1 - my, mz)

        barrier = pltpu.get_barrier_semaphore()
        for nbr in (xpeer, ypeer):
            pl.semaphore_signal(barrier, inc=1, device_id=nbr,
                                device_id_type=pl.DeviceIdType.MESH)
        pl.semaphore_wait(barrier, 2)

        def chunk(ref, t):
            return ref.at[pl.ds(t * c, c), :]

        def direct_id(j):
            return 2 * j + my

        def fwd_id(j):
            return 2 * j + (1 - my)

        xsends = []
        for j in range(J):
            t = direct_id(j)
            rdma = pltpu.make_async_remote_copy(
                src_ref=chunk(xb_ref, t),
                dst_ref=chunk(recv_ref, t),
                send_sem=xs_sems.at[j],
                recv_sem=xr_sems.at[j],
                device_id=xpeer,
                device_id_type=pl.DeviceIdType.MESH,
            )
            rdma.start()
            xsends.append(rdma)

        yrecvs = []
        for j in range(J):
            t = fwd_id(j)
            yrecvs.append(pltpu.make_async_remote_copy(
                src_ref=chunk(recv_ref, t),
                dst_ref=chunk(recv_ref, t),
                send_sem=fs_sems.at[j],
                recv_sem=yr_sems.at[j],
                device_id=ypeer,
                device_id_type=pl.DeviceIdType.MESH,
            ))

        def add_chunk(t):
            cx = pltpu.make_async_copy(chunk(xb_ref, t), xv, cp_sems.at[0])
            cr = pltpu.make_async_copy(chunk(recv_ref, t), rv, cp_sems.at[1])
            cx.start()
            cr.start()
            cx.wait()
            cr.wait()
            ov[...] = xv[...] + rv[...]
            co = pltpu.make_async_copy(ov, chunk(out_ref, t), cp_sems.at[2])
            co.start()
            co.wait()

        fwds = []
        for j in range(J):
            xsends[j].wait_recv()
            t = direct_id(j)
            fwd = pltpu.make_async_remote_copy(
                src_ref=chunk(recv_ref, t),
                dst_ref=chunk(recv_ref, t),
                send_sem=fs_sems.at[j],
                recv_sem=yr_sems.at[j],
                device_id=ypeer,
                device_id_type=pl.DeviceIdType.MESH,
            )
            fwd.start()
            fwds.append(fwd)
            add_chunk(t)
            if j > 0:
                yrecvs[j - 1].wait_recv()
                add_chunk(fwd_id(j - 1))
        yrecvs[J - 1].wait_recv()
        add_chunk(fwd_id(J - 1))

        for j in range(J):
            xsends[j].wait_send()
            fwds[j].wait_send()

    out, _recv = pl.pallas_call(
        body,
        out_shape=(
            jax.ShapeDtypeStruct((m, n), jnp.bfloat16),
            jax.ShapeDtypeStruct((m, n), jnp.bfloat16),
        ),
        in_specs=[pl.BlockSpec(memory_space=pl.ANY)],
        out_specs=(
            pl.BlockSpec(memory_space=pl.ANY),
            pl.BlockSpec(memory_space=pl.ANY),
        ),
        scratch_shapes=[
            pltpu.SemaphoreType.DMA((J,)),
            pltpu.SemaphoreType.DMA((J,)),
            pltpu.SemaphoreType.DMA((J,)),
            pltpu.SemaphoreType.DMA((J,)),
            pltpu.MemorySpace.VMEM((c, n), jnp.bfloat16),
            pltpu.MemorySpace.VMEM((c, n), jnp.bfloat16),
            pltpu.MemorySpace.VMEM((c, n), jnp.bfloat16),
            pltpu.SemaphoreType.DMA((3,)),
        ],
        compiler_params=pltpu.CompilerParams(collective_id=0),
    )(xb)
    return out
